# Initial kernel scaffold; baseline (speedup 1.0000x reference)
import jax
import jax.numpy as jnp
from jax.experimental import pallas as pl


def kernel(input_data, target_data, mask_src, mask_tar):
    raise NotImplementedError("write your pallas kernel here")


# speedup vs baseline: 1096.9310x; 1096.9310x over previous
import jax
import jax.numpy as jnp
from jax import lax
from jax.experimental import pallas as pl
from jax.experimental.pallas import tpu as pltpu
from jax.experimental.pallas import tpu_sc as plsc

H = 512
N = H * H
NC, NS, L = 2, 16, 16
NW = NC * NS
CHUNK = N // NW
VECS = CHUNK // L
NHIST = 9
HB = 256
LH = L * HB
OUTW = NHIST * HB


def _hist_body(inp_hbm, tar_hbm, ma_hbm, mb_hbm, out_hbm,
               inp_v, tar_v, ma_v, mb_v, hist_v, out_v):
    wid = lax.axis_index("s") * NC + lax.axis_index("c")
    base = wid * CHUNK

    for c in range(3):
        pltpu.sync_copy(inp_hbm.at[pl.ds(c * N + base, CHUNK)],
                        inp_v.at[pl.ds(c * CHUNK, CHUNK)])
        pltpu.sync_copy(tar_hbm.at[pl.ds(c * N + base, CHUNK)],
                        tar_v.at[pl.ds(c * CHUNK, CHUNK)])
    pltpu.sync_copy(ma_hbm.at[pl.ds(base, CHUNK)], ma_v)
    pltpu.sync_copy(mb_hbm.at[pl.ds(base, CHUNK)], mb_v)

    zeros = jnp.zeros((L,), jnp.float32)

    def zero_body(j, carry):
        hist_v[pl.ds(j * L, L)] = zeros
        return carry

    lax.fori_loop(0, NHIST * LH // L, zero_body, 0)

    lane = lax.iota(jnp.int32, L) * HB

    def px_body(i, carry):
        off = i * L
        m = ma_v[pl.ds(off, L)]
        mb = mb_v[pl.ds(off, L)]
        for c in range(3):
            x = inp_v[pl.ds(c * CHUNK + off, L)]
            v = jnp.minimum(jnp.maximum((x + 1.0) * 0.5, 0.0), 1.0) * 255.0
            idx = lane + v.astype(jnp.int32)
            plsc.addupdate_scatter(hist_v, [idx + (c * LH)], m)
            plsc.addupdate_scatter(hist_v, [idx + ((3 + c) * LH)], v * m)
            y = tar_v[pl.ds(c * CHUNK + off, L)]
            w = jnp.minimum(jnp.maximum((y + 1.0) * 0.5, 0.0), 1.0) * 255.0
            idx2 = lane + w.astype(jnp.int32)
            plsc.addupdate_scatter(hist_v, [idx2 + ((6 + c) * LH)], mb)
        return carry

    lax.fori_loop(0, VECS, px_body, 0)

    def red_body(j, carry):
        k = j // (HB // L)
        jj = j % (HB // L)
        src = k * LH + jj * L
        acc = hist_v[pl.ds(src, L)]
        for l in range(1, L):
            acc = acc + hist_v[pl.ds(src + l * HB, L)]
        out_v[pl.ds(j * L, L)] = acc
        return carry

    lax.fori_loop(0, NHIST * (HB // L), red_body, 0)

    pltpu.sync_copy(out_v, out_hbm.at[pl.ds(wid * OUTW, OUTW)])


def _make_hist_call(interpret=False):
    mesh = plsc.VectorSubcoreMesh(core_axis_name="c", subcore_axis_name="s",
                                  num_cores=NC, num_subcores=NS)
    return pl.kernel(
        _hist_body,
        out_type=jax.ShapeDtypeStruct((NW * OUTW,), jnp.float32),
        mesh=mesh,
        scratch_types=[
            pltpu.VMEM((3 * CHUNK,), jnp.float32),
            pltpu.VMEM((3 * CHUNK,), jnp.float32),
            pltpu.VMEM((CHUNK,), jnp.float32),
            pltpu.VMEM((CHUNK,), jnp.float32),
            pltpu.VMEM((NHIST * LH,), jnp.float32),
            pltpu.VMEM((OUTW,), jnp.float32),
        ],
        compiler_params=pltpu.CompilerParams(needs_layout_passes=False),
        interpret=interpret,
    )


def kernel(input_data, target_data, mask_src, mask_tar):
    inp = input_data.reshape(3 * N)
    tar = target_data.reshape(3 * N)
    ma = mask_src.reshape(N)
    mb = mask_tar.reshape(N)

    parts = _make_hist_call()(inp, tar, ma, mb)
    hists = parts.reshape(NW, NHIST, HB).sum(axis=0)

    dst_cnt = hists[0:3]
    dst_sum = hists[3:6]
    ref_cnt = hists[6:9]

    total = jnp.float32(0.0)
    idx255 = jnp.arange(1, 256)
    for c in range(3):
        cdf_d = jnp.cumsum(dst_cnt[c] / jnp.sum(dst_cnt[c]))
        cdf_r = jnp.cumsum(ref_cnt[c] / jnp.sum(ref_cnt[c]))
        ref_i = cdf_d[1:256][:, None]
        lo = cdf_r[0:255][None, :]
        hi = cdf_r[1:256][None, :]
        cond = (ref_i >= lo) & (ref_i <= hi)
        any_c = jnp.any(cond, axis=1)
        first_j = jnp.argmax(cond, axis=1) + 1
        vals = jnp.where(any_c, first_j, idx255)
        table = jnp.concatenate([jnp.array([0]), vals])
        table = table.at[255].set(255)
        t = table.astype(jnp.float32)
        total = total + jnp.sum(jnp.abs(dst_sum[c] - dst_cnt[c] * t))

    return total / jnp.float32(3 * N)


# speedup vs baseline: 1143.5994x; 1.0425x over previous
import jax
import jax.numpy as jnp
from jax import lax
from jax.experimental import pallas as pl
from jax.experimental.pallas import tpu as pltpu
from jax.experimental.pallas import tpu_sc as plsc

H = 512
N = H * H
NC, NS, L = 2, 16, 16
NW = NC * NS
CHUNK = N // NW
VECS = CHUNK // L
NHIST = 9
HB = 256
LH = L * HB
OUTW = NHIST * HB
PX_UNROLL = 4


def _hist_body(inp_hbm, tar_hbm, ma_hbm, mb_hbm, out_hbm,
               inp_v, tar_v, ma_v, mb_v, hist_v, out_v, sem):
    wid = lax.axis_index("s") * NC + lax.axis_index("c")
    base = wid * CHUNK

    copies = []
    for c in range(3):
        copies.append(pltpu.async_copy(
            inp_hbm.at[pl.ds(c * N + base, CHUNK)],
            inp_v.at[pl.ds(c * CHUNK, CHUNK)], sem))
        copies.append(pltpu.async_copy(
            tar_hbm.at[pl.ds(c * N + base, CHUNK)],
            tar_v.at[pl.ds(c * CHUNK, CHUNK)], sem))
    copies.append(pltpu.async_copy(ma_hbm.at[pl.ds(base, CHUNK)], ma_v, sem))
    copies.append(pltpu.async_copy(mb_hbm.at[pl.ds(base, CHUNK)], mb_v, sem))

    zeros = jnp.zeros((L,), jnp.float32)

    def zero_body(j, carry):
        for u in range(16):
            hist_v[pl.ds((j * 16 + u) * L, L)] = zeros
        return carry

    lax.fori_loop(0, NHIST * LH // (16 * L), zero_body, 0)

    for cp in copies:
        cp.wait()

    lane = lax.iota(jnp.int32, L)

    def px_body(i, carry):
        for u in range(PX_UNROLL):
            off = (i * PX_UNROLL + u) * L
            m = ma_v[pl.ds(off, L)]
            mb = mb_v[pl.ds(off, L)]
            for c in range(3):
                x = inp_v[pl.ds(c * CHUNK + off, L)]
                v = jnp.minimum(jnp.maximum((x + 1.0) * 0.5, 0.0), 1.0) * 255.0
                idx = lane + v.astype(jnp.int32) * L
                plsc.addupdate_scatter(hist_v, [idx + (c * LH)], m)
                plsc.addupdate_scatter(hist_v, [idx + ((3 + c) * LH)], v * m)
                y = tar_v[pl.ds(c * CHUNK + off, L)]
                w = jnp.minimum(jnp.maximum((y + 1.0) * 0.5, 0.0), 1.0) * 255.0
                idx2 = lane + w.astype(jnp.int32) * L
                plsc.addupdate_scatter(hist_v, [idx2 + ((6 + c) * LH)], mb)
        return carry

    lax.fori_loop(0, VECS // PX_UNROLL, px_body, 0)

    last_lane = lane == (L - 1)
    zero_idx = jnp.zeros((L,), jnp.int32)

    def red_body(j, carry):
        for u in range(8):
            row = j * 8 + u
            s = lax.cumsum(hist_v[pl.ds(row * L, L)], axis=0)
            plsc.store_scatter(out_v, [zero_idx + row], s, mask=last_lane)
        return carry

    lax.fori_loop(0, OUTW // 8, red_body, 0)

    pltpu.sync_copy(out_v, out_hbm.at[pl.ds(wid * OUTW, OUTW)])


def _make_hist_call(interpret=False):
    mesh = plsc.VectorSubcoreMesh(core_axis_name="c", subcore_axis_name="s",
                                  num_cores=NC, num_subcores=NS)
    return pl.kernel(
        _hist_body,
        out_type=jax.ShapeDtypeStruct((NW * OUTW,), jnp.float32),
        mesh=mesh,
        scratch_types=[
            pltpu.VMEM((3 * CHUNK,), jnp.float32),
            pltpu.VMEM((3 * CHUNK,), jnp.float32),
            pltpu.VMEM((CHUNK,), jnp.float32),
            pltpu.VMEM((CHUNK,), jnp.float32),
            pltpu.VMEM((NHIST * LH,), jnp.float32),
            pltpu.VMEM((OUTW,), jnp.float32),
            pltpu.SemaphoreType.DMA,
        ],
        compiler_params=pltpu.CompilerParams(needs_layout_passes=False),
        interpret=interpret,
    )


def kernel(input_data, target_data, mask_src, mask_tar):
    inp = input_data.reshape(3 * N)
    tar = target_data.reshape(3 * N)
    ma = mask_src.reshape(N)
    mb = mask_tar.reshape(N)

    parts = _make_hist_call()(inp, tar, ma, mb)
    hists = parts.reshape(NW, NHIST, HB).sum(axis=0)

    dst_cnt = hists[0:3]
    dst_sum = hists[3:6]
    ref_cnt = hists[6:9]

    cdf_d = jnp.stack([jnp.cumsum(dst_cnt[c] / jnp.sum(dst_cnt[c]))
                       for c in range(3)])
    cdf_r = jnp.stack([jnp.cumsum(ref_cnt[c] / jnp.sum(ref_cnt[c]))
                       for c in range(3)])

    cond = ((cdf_d[:, 1:, None] >= cdf_r[:, None, 0:255])
            & (cdf_d[:, 1:, None] <= cdf_r[:, None, 1:256]))
    any_c = jnp.any(cond, axis=2)
    first_j = jnp.argmax(cond, axis=2) + 1
    vals = jnp.where(any_c, first_j, jnp.arange(1, 256)[None, :])
    table = jnp.concatenate(
        [jnp.zeros((3, 1), vals.dtype), vals], axis=1).at[:, 255].set(255)
    t = table.astype(jnp.float32)

    return jnp.sum(jnp.abs(dst_sum - dst_cnt * t)) / jnp.float32(3 * N)


# speedup vs baseline: 1386.2346x; 1.2122x over previous
import jax
import jax.numpy as jnp
from jax import lax
from jax.experimental import pallas as pl
from jax.experimental.pallas import tpu as pltpu
from jax.experimental.pallas import tpu_sc as plsc

H = 512
N = H * H
NC, NS, L = 2, 16, 16
NW = NC * NS
CHUNK = N // NW
VECS = CHUNK // L
NHIST = 9
HB = 256
STRIDE = HB + 1
HWORDS = (L - 1) * STRIDE + HB
HALLOC = 4112
OUTW = NHIST * HB
PX_UNROLL = 4


def _hist_body(inp_hbm, tar_hbm, ma_hbm, mb_hbm, out_hbm,
               inp_v, tar_v, ma_v, mb_v,
               h0, h1, h2, h3, h4, h5, h6, h7, h8, out_v, sem):
    hs = [h0, h1, h2, h3, h4, h5, h6, h7, h8]
    wid = lax.axis_index("s") * NC + lax.axis_index("c")
    base = wid * CHUNK

    copies = []
    for c in range(3):
        copies.append(pltpu.async_copy(
            inp_hbm.at[pl.ds(c * N + base, CHUNK)],
            inp_v.at[pl.ds(c * CHUNK, CHUNK)], sem))
        copies.append(pltpu.async_copy(
            tar_hbm.at[pl.ds(c * N + base, CHUNK)],
            tar_v.at[pl.ds(c * CHUNK, CHUNK)], sem))
    copies.append(pltpu.async_copy(ma_hbm.at[pl.ds(base, CHUNK)], ma_v, sem))
    copies.append(pltpu.async_copy(mb_hbm.at[pl.ds(base, CHUNK)], mb_v, sem))

    zeros = jnp.zeros((L,), jnp.float32)

    def zero_body(j, carry):
        for h in hs:
            h[pl.ds(j * L, L)] = zeros
        return carry

    lax.fori_loop(0, HALLOC // L, zero_body, 0)

    for cp in copies:
        cp.wait()

    lane = lax.iota(jnp.int32, L) * STRIDE

    def px_body(i, carry):
        for u in range(PX_UNROLL):
            off = (i * PX_UNROLL + u) * L
            m = ma_v[pl.ds(off, L)]
            mb = mb_v[pl.ds(off, L)]
            for c in range(3):
                x = inp_v[pl.ds(c * CHUNK + off, L)]
                v = jnp.minimum(jnp.maximum((x + 1.0) * 0.5, 0.0), 1.0) * 255.0
                idx = lane + v.astype(jnp.int32)
                plsc.addupdate_scatter(hs[c], [idx], m)
                plsc.addupdate_scatter(hs[3 + c], [idx], v * m)
                y = tar_v[pl.ds(c * CHUNK + off, L)]
                w = jnp.minimum(jnp.maximum((y + 1.0) * 0.5, 0.0), 1.0) * 255.0
                idx2 = lane + w.astype(jnp.int32)
                plsc.addupdate_scatter(hs[6 + c], [idx2], mb)
        return carry

    lax.fori_loop(0, VECS // PX_UNROLL, px_body, 0)

    def red_body(j, carry):
        for k in range(NHIST):
            acc = hs[k][pl.ds(j * L, L)]
            for l in range(1, L):
                acc = acc + hs[k][pl.ds(l * STRIDE + j * L, L)]
            out_v[pl.ds(k * HB + j * L, L)] = acc
        return carry

    lax.fori_loop(0, HB // L, red_body, 0)

    pltpu.sync_copy(out_v, out_hbm.at[pl.ds(wid * OUTW, OUTW)])


def _make_hist_call(interpret=False):
    mesh = plsc.VectorSubcoreMesh(core_axis_name="c", subcore_axis_name="s",
                                  num_cores=NC, num_subcores=NS)
    return pl.kernel(
        _hist_body,
        out_type=jax.ShapeDtypeStruct((NW * OUTW,), jnp.float32),
        mesh=mesh,
        scratch_types=[
            pltpu.VMEM((3 * CHUNK,), jnp.float32),
            pltpu.VMEM((3 * CHUNK,), jnp.float32),
            pltpu.VMEM((CHUNK,), jnp.float32),
            pltpu.VMEM((CHUNK,), jnp.float32),
        ] + [pltpu.VMEM((HALLOC,), jnp.float32) for _ in range(NHIST)] + [
            pltpu.VMEM((OUTW,), jnp.float32),
            pltpu.SemaphoreType.DMA,
        ],
        compiler_params=pltpu.CompilerParams(needs_layout_passes=False),
        interpret=interpret,
    )


def kernel(input_data, target_data, mask_src, mask_tar):
    inp = input_data.reshape(3 * N)
    tar = target_data.reshape(3 * N)
    ma = mask_src.reshape(N)
    mb = mask_tar.reshape(N)

    parts = _make_hist_call()(inp, tar, ma, mb)
    hists = parts.reshape(NW, NHIST, HB).sum(axis=0)

    dst_cnt = hists[0:3]
    dst_sum = hists[3:6]
    ref_cnt = hists[6:9]

    cdf_d = jnp.stack([jnp.cumsum(dst_cnt[c] / jnp.sum(dst_cnt[c]))
                       for c in range(3)])
    cdf_r = jnp.stack([jnp.cumsum(ref_cnt[c] / jnp.sum(ref_cnt[c]))
                       for c in range(3)])

    cond = ((cdf_d[:, 1:, None] >= cdf_r[:, None, 0:255])
            & (cdf_d[:, 1:, None] <= cdf_r[:, None, 1:256]))
    any_c = jnp.any(cond, axis=2)
    first_j = jnp.argmax(cond, axis=2) + 1
    vals = jnp.where(any_c, first_j, jnp.arange(1, 256)[None, :])
    table = jnp.concatenate(
        [jnp.zeros((3, 1), vals.dtype), vals], axis=1).at[:, 255].set(255)
    t = table.astype(jnp.float32)

    return jnp.sum(jnp.abs(dst_sum - dst_cnt * t)) / jnp.float32(3 * N)
